# baseline (speedup 1.0000x reference)
import functools

import jax
import jax.numpy as jnp
from jax import lax
from jax.experimental import pallas as pl
from jax.experimental.pallas import tpu as pltpu
from jax.experimental.pallas import tpu_sc as plsc

_PAD = 1
_NC, _NS = 2, 16
_NW = _NC * _NS
_TOK = 16
_NST = 2


@functools.lru_cache(maxsize=None)
def _build(bsz, seq, vocab, dim):
    b_total = bsz * seq
    span = seq // _NW
    n_chunks = span // _TOK
    assert span * _NW == seq and n_chunks * _TOK == span
    assert n_chunks % _NST == 0
    assert vocab >= seq + _PAD + 1
    assert dim % 8 == 0

    mesh = plsc.VectorSubcoreMesh(
        core_axis_name="c", subcore_axis_name="s",
        num_cores=_NC, num_subcores=_NS,
    )

    @functools.partial(
        pl.kernel,
        out_type=jax.ShapeDtypeStruct((b_total * dim,), jnp.float32),
        mesh=mesh,
        scratch_types=[
            pltpu.VMEM((b_total,), jnp.int32),
            pltpu.VMEM((bsz * span,), jnp.int32),
            pltpu.SMEM((bsz * n_chunks,), jnp.int32),
            pltpu.VMEM((_TOK * dim,), jnp.float32),
        ]
        + [pltpu.VMEM((_TOK * dim,), jnp.float32) for _ in range(_NST)]
        + [pltpu.SemaphoreType.DMA for _ in range(2 * _NST + 1)],
        compiler_params=pltpu.CompilerParams(needs_layout_passes=False),
    )
    def sc_all(inp_hbm, tab_hbm, out_hbm, inp_v, pos_v, pfx_s, slow_v, *rest):
        stg = rest[:_NST]
        gsems = rest[_NST : 2 * _NST]
        ssems = rest[2 * _NST : 3 * _NST]
        slowsem = rest[3 * _NST]
        wid = lax.axis_index("s") * _NC + lax.axis_index("c")
        t0w = wid * span

        pltpu.sync_copy(inp_hbm, inp_v)

        for b in range(bsz):
            def count_step(i, acc, b=b):
                off = pl.multiple_of(i * 16, 16)
                x = inp_v[pl.ds(b * seq + off, 16)]
                return acc + jnp.where(x != _PAD, 1, 0).astype(jnp.int32)

            acc = lax.fori_loop(
                0, t0w // 16, count_step, jnp.zeros((16,), jnp.int32)
            )
            prefix0 = jnp.sum(acc)

            def scan_step(c, prefix, b=b):
                off = pl.multiple_of(c * 16, 16)
                x = inp_v[pl.ds(b * seq + t0w + off, 16)]
                m = jnp.where(x != _PAD, 1, 0).astype(jnp.int32)
                cs = plsc.cumsum(m)
                pos_v[pl.ds(b * span + off, 16)] = (prefix + cs) * m + _PAD
                nxt = prefix + jnp.sum(m)
                pfx_s[b * n_chunks + c] = nxt
                return nxt

            lax.fori_loop(0, n_chunks, scan_step, prefix0)

        def start_stage(c, k):
            src = pl.multiple_of((t0w + c * _TOK + 1 + _PAD) * dim, 8)
            pltpu.async_copy(tab_hbm.at[pl.ds(src, _TOK * dim)], stg[k], gsems[k])

        def wait_stage(k):
            pltpu.make_async_copy(
                tab_hbm.at[pl.ds(0, _TOK * dim)], stg[k], gsems[k]
            ).wait()

        def fan_out(c, k):
            toff = pl.multiple_of(c * _TOK, _TOK)
            for b in range(bsz):
                fast = pfx_s[b * n_chunks + c] == t0w + (c + 1) * _TOK
                dst = pl.multiple_of((b * seq + t0w + toff) * dim, 8)

                @pl.when(fast)
                def _(dst=dst, k=k):
                    pltpu.async_copy(
                        stg[k], out_hbm.at[pl.ds(dst, _TOK * dim)], ssems[k]
                    )

                @pl.when(jnp.logical_not(fast))
                def _(dst=dst, b=b, toff=toff):
                    p_vec = pos_v[pl.ds(b * span + toff, _TOK)]
                    for j in range(_TOK):
                        src_j = pl.multiple_of(p_vec[j] * dim, 8)
                        pltpu.async_copy(
                            tab_hbm.at[pl.ds(src_j, dim)],
                            slow_v.at[pl.ds(j * dim, dim)],
                            slowsem,
                        )
                    pltpu.make_async_copy(
                        tab_hbm.at[pl.ds(0, _TOK * dim)], slow_v, slowsem
                    ).wait()
                    cp = pltpu.async_copy(
                        slow_v, out_hbm.at[pl.ds(dst, _TOK * dim)], slowsem
                    )
                    cp.wait()

        def drain_fanout(c, k):
            for b in range(bsz):
                fast = pfx_s[b * n_chunks + c] == t0w + (c + 1) * _TOK

                @pl.when(fast)
                def _(k=k):
                    pltpu.make_async_copy(
                        stg[k], out_hbm.at[pl.ds(0, _TOK * dim)], ssems[k]
                    ).wait()

        n_groups = n_chunks // _NST
        for k in range(_NST):
            start_stage(k, k)

        def group(g, _):
            c0 = g * _NST
            for k in range(_NST):
                wait_stage(k)
                fan_out(c0 + k, k)
            for k in range(_NST):
                drain_fanout(c0 + k, k)

                @pl.when(g + 1 < n_groups)
                def _(k=k, c0=c0):
                    start_stage(c0 + _NST + k, k)

            return 0

        lax.fori_loop(0, n_groups, group, 0)

    def run(inp, weights):
        flat = sc_all(inp.reshape(b_total), weights.reshape(vocab * dim))
        return flat.reshape(bsz, seq, dim)

    return run


@jax.jit
def kernel(input, weights):
    bsz, seq = input.shape
    vocab, dim = weights.shape
    run = _build(bsz, seq, vocab, dim)
    return run(input.astype(jnp.int32), weights.astype(jnp.float32))


# speedup vs baseline: 2.2448x; 2.2448x over previous
import functools

import jax
import jax.numpy as jnp
from jax import lax
from jax.experimental import pallas as pl
from jax.experimental.pallas import tpu as pltpu
from jax.experimental.pallas import tpu_sc as plsc

_PAD = 1
_NC, _NS = 2, 16
_NW = _NC * _NS
_CHUNK = 16
_NBUF = 4


@functools.lru_cache(maxsize=None)
def _build(bsz, seq, vocab, dim):
    b_total = bsz * seq
    b_per_w = b_total // _NW
    n_chunks = b_per_w // _CHUNK
    assert b_per_w * _NW == b_total and n_chunks * _CHUNK == b_per_w
    assert seq % b_per_w == 0 and n_chunks % _NBUF == 0

    w_per_row = seq // b_per_w

    mesh = plsc.VectorSubcoreMesh(
        core_axis_name="c", subcore_axis_name="s",
        num_cores=_NC, num_subcores=_NS,
    )

    @functools.partial(
        pl.kernel,
        out_type=jax.ShapeDtypeStruct((b_total, dim), jnp.float32),
        mesh=mesh,
        scratch_types=[
            pltpu.VMEM((seq,), jnp.int32),
            pltpu.VMEM((b_per_w,), jnp.int32),
        ]
        + [pltpu.VMEM((_CHUNK, dim), jnp.float32) for _ in range(_NBUF)]
        + [pltpu.SemaphoreType.DMA for _ in range(2 * _NBUF)],
        compiler_params=pltpu.CompilerParams(needs_layout_passes=False),
    )
    def sc_all(inp_hbm, table_hbm, out_hbm, row_v, pos_v, *rest):
        bufs = rest[:_NBUF]
        gsems = rest[_NBUF : 2 * _NBUF]
        ssems = rest[2 * _NBUF : 3 * _NBUF]
        wid = lax.axis_index("s") * _NC + lax.axis_index("c")
        base = wid * b_per_w
        row = wid // w_per_row
        s0 = (wid % w_per_row) * b_per_w

        pltpu.sync_copy(inp_hbm.at[pl.ds(row * seq, seq)], row_v)

        def count_step(i, acc):
            off = pl.multiple_of(i * 16, 16)
            x = row_v[pl.ds(off, 16)]
            return acc + jnp.where(x != _PAD, 1, 0).astype(jnp.int32)

        acc = lax.fori_loop(
            0, s0 // 16, count_step, jnp.zeros((16,), jnp.int32)
        )
        prefix0 = jnp.sum(acc)

        def scan_step(i, prefix):
            off = pl.multiple_of(i * 16, 16)
            x = row_v[pl.ds(s0 + off, 16)]
            m = jnp.where(x != _PAD, 1, 0).astype(jnp.int32)
            c = plsc.cumsum(m)
            pos_v[pl.ds(off, 16)] = (prefix + c) * m + _PAD
            return prefix + jnp.sum(m)

        lax.fori_loop(0, b_per_w // 16, scan_step, prefix0)

        def start_g(i, k):
            off = pl.multiple_of(i * _CHUNK, _CHUNK)
            pltpu.async_copy(
                table_hbm.at[pos_v.at[pl.ds(off, _CHUNK)]], bufs[k], gsems[k]
            )

        def start_s(i, k):
            off = pl.multiple_of(i * _CHUNK, _CHUNK)
            pltpu.async_copy(
                bufs[k], out_hbm.at[pl.ds(base + off, _CHUNK)], ssems[k]
            )

        def wait_g(k):
            pltpu.make_async_copy(
                table_hbm.at[pos_v.at[pl.ds(0, _CHUNK)]], bufs[k], gsems[k]
            ).wait()

        def wait_s(k):
            pltpu.make_async_copy(
                bufs[k], out_hbm.at[pl.ds(base, _CHUNK)], ssems[k]
            ).wait()

        n_groups = n_chunks // _NBUF
        for k in range(_NBUF):
            start_g(k, k)

        def group(g, _):
            i0 = g * _NBUF
            for k in range(_NBUF):
                wait_g(k)
                start_s(i0 + k, k)
            for k in range(_NBUF):
                wait_s(k)

                @pl.when(g + 1 < n_groups)
                def _(k=k, i0=i0):
                    start_g(i0 + _NBUF + k, k)

            return 0

        lax.fori_loop(0, n_groups, group, 0)

    def run(inp, weights):
        flat = sc_all(inp.reshape(b_total), weights)
        return flat.reshape(bsz, seq, dim)

    return run


@jax.jit
def kernel(input, weights):
    bsz, seq = input.shape
    vocab, dim = weights.shape
    run = _build(bsz, seq, vocab, dim)
    return run(input.astype(jnp.int32), weights.astype(jnp.float32))
